# baseline (speedup 1.0000x reference)
import functools

import jax
import jax.numpy as jnp
from jax import lax
from jax.experimental import pallas as pl
from jax.experimental.pallas import tpu as pltpu
from jax.experimental.pallas import tpu_sc as plsc

NUM_CORES = 2
NUM_SUBCORES = 16
NUM_WORKERS = NUM_CORES * NUM_SUBCORES
CHUNK_ROWS = 64


@functools.lru_cache(maxsize=None)
def _make_sc_broadcast(batch: int, seq_len: int, d_model: int):
    rows_per_worker = seq_len // NUM_WORKERS
    n_chunks = rows_per_worker // CHUNK_ROWS
    assert rows_per_worker % CHUNK_ROWS == 0

    mesh = plsc.VectorSubcoreMesh(
        core_axis_name="c", subcore_axis_name="s",
        num_cores=NUM_CORES, num_subcores=NUM_SUBCORES,
    )

    @functools.partial(
        pl.kernel,
        out_type=jax.ShapeDtypeStruct((batch, seq_len, d_model), jnp.float32),
        mesh=mesh,
        scratch_types=[
            pltpu.VMEM((2, CHUNK_ROWS, d_model), jnp.float32),
            pltpu.SemaphoreType.DMA,
            pltpu.SemaphoreType.DMA,
        ],
    )
    def table_broadcast(table_hbm, out_hbm, buf, in_sem, out_sem):
        cid = lax.axis_index("c")
        wid = lax.axis_index("s") * NUM_CORES + cid
        base = wid * rows_per_worker

        pltpu.async_copy(
            table_hbm.at[pl.ds(base, CHUNK_ROWS)], buf.at[0], in_sem)

        for c in range(n_chunks):
            cur = c % 2
            nxt = (c + 1) % 2
            if c + 1 < n_chunks:
                if c >= 1:
                    for b in range(batch):
                        pltpu.make_async_copy(
                            buf.at[nxt],
                            out_hbm.at[b, pl.ds(base, CHUNK_ROWS)], out_sem,
                        ).wait()
                pltpu.async_copy(
                    table_hbm.at[pl.ds(base + (c + 1) * CHUNK_ROWS, CHUNK_ROWS)],
                    buf.at[nxt], in_sem)
            pltpu.make_async_copy(
                table_hbm.at[pl.ds(base, CHUNK_ROWS)], buf.at[cur], in_sem
            ).wait()
            for b in range(batch):
                bb = lax.rem(b + wid, batch)
                pltpu.async_copy(
                    buf.at[cur],
                    out_hbm.at[bb, pl.ds(base + c * CHUNK_ROWS, CHUNK_ROWS)],
                    out_sem)
        for c in range(max(0, n_chunks - 2), n_chunks):
            cur = c % 2
            for b in range(batch):
                pltpu.make_async_copy(
                    buf.at[cur],
                    out_hbm.at[b, pl.ds(base, CHUNK_ROWS)], out_sem,
                ).wait()

    return table_broadcast


def kernel(x, pos_table):
    batch, seq_len = x.shape
    d_model = pos_table.shape[1]
    return _make_sc_broadcast(batch, seq_len, d_model)(pos_table)


# speedup vs baseline: 1.0167x; 1.0167x over previous
import functools

import jax
import jax.numpy as jnp
from jax import lax
from jax.experimental import pallas as pl
from jax.experimental.pallas import tpu as pltpu
from jax.experimental.pallas import tpu_sc as plsc

NUM_CORES = 2
NUM_SUBCORES = 16
CHUNK_ROWS = 64
SP_ROWS = 1024
SP_CHUNK = 256


@functools.lru_cache(maxsize=None)
def _make_sc_broadcast(batch: int, seq_len: int, d_model: int):
    rows_per_core = seq_len // NUM_CORES
    stream_rows = rows_per_core - SP_ROWS
    rows_per_worker = stream_rows // NUM_SUBCORES
    n_chunks = rows_per_worker // CHUNK_ROWS
    n_sp_chunks = SP_ROWS // SP_CHUNK
    assert rows_per_worker % CHUNK_ROWS == 0 and SP_ROWS % SP_CHUNK == 0

    mesh = plsc.VectorSubcoreMesh(
        core_axis_name="c", subcore_axis_name="s",
        num_cores=NUM_CORES, num_subcores=NUM_SUBCORES,
    )

    @functools.partial(
        pl.kernel,
        out_type=jax.ShapeDtypeStruct((batch, seq_len, d_model), jnp.float32),
        mesh=mesh,
        scratch_types=[
            pltpu.VMEM((2, CHUNK_ROWS, d_model), jnp.float32),
            pltpu.VMEM_SHARED((2, SP_CHUNK, d_model), jnp.float32),
            pltpu.SemaphoreType.DMA,
            pltpu.SemaphoreType.DMA,
            pltpu.SemaphoreType.DMA,
            pltpu.SemaphoreType.DMA,
        ],
    )
    def table_broadcast(table_hbm, out_hbm, buf, spbuf,
                        in_sem, out_sem, sp_in_sem, sp_out_sem):
        cid = lax.axis_index("c")
        sid = lax.axis_index("s")
        base = cid * rows_per_core + sid * rows_per_worker
        sp_base = cid * rows_per_core + stream_rows

        def sp_step(k):
            @pl.when(sid == 0)
            def _():
                cur = k % 2
                nxt = (k + 1) % 2
                if k + 1 < n_sp_chunks:
                    if k >= 1:
                        for b in range(batch):
                            pltpu.make_async_copy(
                                spbuf.at[nxt],
                                out_hbm.at[b, pl.ds(sp_base, SP_CHUNK)],
                                sp_out_sem,
                            ).wait()
                    pltpu.async_copy(
                        table_hbm.at[pl.ds(sp_base + (k + 1) * SP_CHUNK,
                                           SP_CHUNK)],
                        spbuf.at[nxt], sp_in_sem)
                pltpu.make_async_copy(
                    table_hbm.at[pl.ds(sp_base, SP_CHUNK)], spbuf.at[cur],
                    sp_in_sem,
                ).wait()
                for b in range(batch):
                    pltpu.async_copy(
                        spbuf.at[cur],
                        out_hbm.at[b, pl.ds(sp_base + k * SP_CHUNK, SP_CHUNK)],
                        sp_out_sem)

        @pl.when(sid == 0)
        def _sp_prime():
            pltpu.async_copy(
                table_hbm.at[pl.ds(sp_base, SP_CHUNK)], spbuf.at[0], sp_in_sem)

        pltpu.async_copy(
            table_hbm.at[pl.ds(base, CHUNK_ROWS)], buf.at[0], in_sem)

        sp_next = 0
        for c in range(n_chunks):
            cur = c % 2
            nxt = (c + 1) % 2
            if c + 1 < n_chunks:
                if c >= 1:
                    for b in range(batch):
                        pltpu.make_async_copy(
                            buf.at[nxt],
                            out_hbm.at[b, pl.ds(base, CHUNK_ROWS)], out_sem,
                        ).wait()
                pltpu.async_copy(
                    table_hbm.at[pl.ds(base + (c + 1) * CHUNK_ROWS, CHUNK_ROWS)],
                    buf.at[nxt], in_sem)
            pltpu.make_async_copy(
                table_hbm.at[pl.ds(base, CHUNK_ROWS)], buf.at[cur], in_sem
            ).wait()
            for b in range(batch):
                pltpu.async_copy(
                    buf.at[cur],
                    out_hbm.at[b, pl.ds(base + c * CHUNK_ROWS, CHUNK_ROWS)],
                    out_sem)
            if sp_next < n_sp_chunks:
                sp_step(sp_next)
                sp_next += 1
        while sp_next < n_sp_chunks:
            sp_step(sp_next)
            sp_next += 1

        for c in range(max(0, n_chunks - 2), n_chunks):
            cur = c % 2
            for b in range(batch):
                pltpu.make_async_copy(
                    buf.at[cur],
                    out_hbm.at[b, pl.ds(base, CHUNK_ROWS)], out_sem,
                ).wait()

        @pl.when(sid == 0)
        def _sp_drain():
            for k in range(max(0, n_sp_chunks - 2), n_sp_chunks):
                cur = k % 2
                for b in range(batch):
                    pltpu.make_async_copy(
                        spbuf.at[cur],
                        out_hbm.at[b, pl.ds(sp_base, SP_CHUNK)], sp_out_sem,
                    ).wait()

    return table_broadcast


def kernel(x, pos_table):
    batch, seq_len = x.shape
    d_model = pos_table.shape[1]
    return _make_sc_broadcast(batch, seq_len, d_model)(pos_table)
